# Initial kernel scaffold; baseline (speedup 1.0000x reference)
import jax
import jax.numpy as jnp
from jax.experimental import pallas as pl


def kernel(imgs, words, table, Wih_f, Whh_f, b_f, Wih_b, Whh_b, b_b, Wu1, Wfu, bfu, Wu2, We1, We2, We3, Wfe, bfe, W1, b1, W2, b2):
    raise NotImplementedError("write your pallas kernel here")


# speedup vs baseline: 1.3069x; 1.3069x over previous
import functools

import jax
import jax.numpy as jnp
from jax import lax
from jax.experimental import pallas as pl
from jax.experimental.pallas import tpu as pltpu
from jax.experimental.pallas import tpu_sc as plsc

F32 = jnp.float32


def _bf(v):
    return v.astype(jnp.bfloat16).astype(F32)


def _sc_gather(table, idxp):
    info = plsc.get_sparse_core_info()
    nc, ns = info.num_cores, info.num_subcores
    nw = nc * ns
    bpw = 256 // nw
    mesh = plsc.VectorSubcoreMesh(core_axis_name="c", subcore_axis_name="s")

    @functools.partial(
        pl.kernel, mesh=mesh,
        out_type=jax.ShapeDtypeStruct((256, 512), F32),
        scratch_types=[
            pltpu.VMEM((bpw,), jnp.int32),
            pltpu.VMEM((bpw, 512), F32),
            pltpu.SemaphoreType.DMA,
        ],
    )
    def k(table_hbm, idx_hbm, out_hbm, idx_v, rows_v, sem):
        wid = lax.axis_index("s") * nc + lax.axis_index("c")
        base = wid * bpw
        pltpu.sync_copy(idx_hbm.at[pl.ds(base, bpw)], idx_v)
        pltpu.async_copy(table_hbm.at[idx_v], rows_v, sem).wait()
        pltpu.sync_copy(rows_v, out_hbm.at[pl.ds(base, bpw)])

    return k(table, idxp)


def _lstm_body(x_ref, wihf_ref, wihb_ref, bcat_ref, whhf_ref, whhb_ref,
               wfu_ref, bfu_ref, wfe_ref, bfe_ref,
               q_ref, gb_ref, gb2_ref, xw_ref):
    x = x_ref[...]
    dn = (((1,), (1,)), ((), ()))
    xwf = lax.dot_general(x, wihf_ref[...], dn, preferred_element_type=F32)
    xwb = lax.dot_general(x, wihb_ref[...], dn, preferred_element_type=F32)
    xw_ref[...] = jnp.concatenate([xwf, xwb], axis=1) + bcat_ref[...]

    whhf = whhf_ref[...]
    whhb = whhb_ref[...]

    def step(t, carry):
        hf, cf, hb, cb, sf, sb = carry
        gf = xw_ref[pl.ds(t * 8, 8), :1024] + lax.dot_general(
            hf, whhf, dn, preferred_element_type=F32)
        i = jax.nn.sigmoid(gf[:, :256])
        f = jax.nn.sigmoid(gf[:, 256:512])
        g = jnp.tanh(gf[:, 512:768])
        o = jax.nn.sigmoid(gf[:, 768:])
        cf = f * cf + i * g
        hf = o * jnp.tanh(cf)
        gb_ = xw_ref[pl.ds((19 - t) * 8, 8), 1024:] + lax.dot_general(
            hb, whhb, dn, preferred_element_type=F32)
        i = jax.nn.sigmoid(gb_[:, :256])
        f = jax.nn.sigmoid(gb_[:, 256:512])
        g = jnp.tanh(gb_[:, 512:768])
        o = jax.nn.sigmoid(gb_[:, 768:])
        cb = f * cb + i * g
        hb = o * jnp.tanh(cb)
        return hf, cf, hb, cb, sf + hf, sb + hb

    z = jnp.zeros((8, 256), F32)
    hf, cf, hb, cb, sf, sb = lax.fori_loop(0, 20, step, (z, z, z, z, z, z))
    q_ref[...] = jnp.concatenate([hf, hb], axis=1)
    cond = jnp.concatenate([sf, sb], axis=1) * (1.0 / 20.0)
    gb_ref[...] = jnp.dot(cond, wfu_ref[...], preferred_element_type=F32) + bfu_ref[...]
    gb2_ref[...] = jnp.dot(cond, wfe_ref[...], preferred_element_type=F32) + bfe_ref[...]


def _lstm(x, Wih_f, Wih_b, bcat, Whh_f, Whh_b, Wfu, bfu2, Wfe, bfe2):
    return pl.pallas_call(
        _lstm_body,
        out_shape=(
            jax.ShapeDtypeStruct((8, 512), F32),
            jax.ShapeDtypeStruct((8, 64), F32),
            jax.ShapeDtypeStruct((8, 512), F32),
        ),
        scratch_shapes=[pltpu.VMEM((160, 2048), F32)],
    )(x, Wih_f, Wih_b, bcat, Whh_f, Whh_b, Wfu, bfu2, Wfe, bfe2)


def _shift(p, sy, sx):
    if sy == 1:
        p = jnp.concatenate([p[..., 1:, :], jnp.zeros_like(p[..., :1, :])], axis=-2)
    elif sy == -1:
        p = jnp.concatenate([jnp.zeros_like(p[..., :1, :]), p[..., :-1, :]], axis=-2)
    if sx == 1:
        p = jnp.concatenate([p[..., :, 1:], jnp.zeros_like(p[..., :, :1])], axis=-1)
    elif sx == -1:
        p = jnp.concatenate([jnp.zeros_like(p[..., :, :1]), p[..., :, :-1]], axis=-1)
    return p


_TAP1 = ((0, 0), (1, 0), (0, 1))


def _conv1_body(ph_ref, w_ref, gamma_ref, beta_ref, h1_ref):
    phb = _bf(ph_ref[0])
    wb = _bf(w_ref[...])
    acc = jnp.zeros((32, 112, 112), F32)
    for ci in range(3):
        for dy in range(3):
            py, sy = _TAP1[dy]
            for dx in range(3):
                px, sx = _TAP1[dx]
                pln = _shift(phb[ci, py, px], sy, sx)
                wv = wb[:, ci * 9 + dy * 3 + dx].reshape(32, 1, 1)
                acc = acc + wv * pln[None]
    acc = jnp.maximum(acc, 0.0)
    gamma = gamma_ref[0].reshape(32, 1, 1)
    beta = beta_ref[0].reshape(32, 1, 1)
    h1_ref[0] = acc * (1.0 + gamma) + beta


def _conv1(ph, w1r, gamma3, beta3):
    return pl.pallas_call(
        _conv1_body,
        grid=(8,),
        in_specs=[
            pl.BlockSpec((1, 3, 2, 2, 112, 112), lambda b: (b, 0, 0, 0, 0, 0)),
            pl.BlockSpec((32, 27), lambda b: (0, 0)),
            pl.BlockSpec((1, 1, 32), lambda b: (b, 0, 0)),
            pl.BlockSpec((1, 1, 32), lambda b: (b, 0, 0)),
        ],
        out_specs=pl.BlockSpec((1, 32, 112, 112), lambda b: (b, 0, 0, 0)),
        out_shape=jax.ShapeDtypeStruct((8, 32, 112, 112), F32),
    )(ph, w1r, gamma3, beta3)


_CMAP = {0: ((1, -1), (0, 0), (1, 0)), 1: ((0, 0), (1, 0), (0, 1))}


def _updown_body(h1_ref, ph_ref, w_ref, wph_ref, mph_ref):
    x = h1_ref[0]
    rsh = jnp.concatenate([x[:, :, :1], x[:, :, :-1]], axis=2)
    lsh = jnp.concatenate([x[:, :, 1:], x[:, :, -1:]], axis=2)
    upc = (0.75 * x + 0.25 * rsh, 0.75 * x + 0.25 * lsh)
    up = {}
    for px in range(2):
        u = upc[px]
        rshr = jnp.concatenate([u[:, :1, :], u[:, :-1, :]], axis=1)
        lshr = jnp.concatenate([u[:, 1:, :], u[:, -1:, :]], axis=1)
        up[(0, px)] = _bf(0.75 * u + 0.25 * rshr)
        up[(1, px)] = _bf(0.75 * u + 0.25 * lshr)
    wb = _bf(w_ref[...])
    for py in range(2):
        for px in range(2):
            acc = jnp.zeros((32, 112, 112), F32)
            for dy in range(3):
                ry, sy = _CMAP[py][dy]
                for dx in range(3):
                    rx, sx = _CMAP[px][dx]
                    pln = _shift(up[(ry, rx)], sy, sx)
                    acc = acc + wb[:, dy * 3 + dx].reshape(32, 1, 1) * pln
            wpl = jnp.sum(acc, axis=0)
            wph_ref[0, py, px] = wpl
            m = (wpl > 0.0).astype(F32)
            for ci in range(3):
                mph_ref[0, ci, py, px] = ph_ref[0, ci, py, px] * m


def _updown(h1, ph, w2r):
    return pl.pallas_call(
        _updown_body,
        grid=(8,),
        in_specs=[
            pl.BlockSpec((1, 32, 112, 112), lambda b: (b, 0, 0, 0)),
            pl.BlockSpec((1, 3, 2, 2, 112, 112), lambda b: (b, 0, 0, 0, 0, 0)),
            pl.BlockSpec((32, 9), lambda b: (0, 0)),
        ],
        out_specs=(
            pl.BlockSpec((1, 2, 2, 112, 112), lambda b: (b, 0, 0, 0, 0)),
            pl.BlockSpec((1, 3, 2, 2, 112, 112), lambda b: (b, 0, 0, 0, 0, 0)),
        ),
        out_shape=(
            jax.ShapeDtypeStruct((8, 2, 2, 112, 112), F32),
            jax.ShapeDtypeStruct((8, 3, 2, 2, 112, 112), F32),
        ),
    )(h1, ph, w2r)


def _mm_relu_body(x_ref, w_ref, o_ref):
    o_ref[...] = jnp.maximum(
        jnp.dot(x_ref[...], w_ref[...], preferred_element_type=F32), 0.0)


def _mm_relu_grid(x, w, mblk):
    m, k = x.shape
    n = w.shape[1]
    return pl.pallas_call(
        _mm_relu_body,
        grid=(m // mblk,),
        in_specs=[
            pl.BlockSpec((mblk, k), lambda i: (i, 0)),
            pl.BlockSpec((k, n), lambda i: (0, 0)),
        ],
        out_specs=pl.BlockSpec((mblk, n), lambda i: (i, 0)),
        out_shape=jax.ShapeDtypeStruct((m, n), F32),
    )(x, w)


def _mm_film_body(x_ref, w_ref, g_ref, b_ref, o_ref):
    e = jnp.dot(x_ref[...], w_ref[...], preferred_element_type=F32)
    o_ref[...] = jnp.maximum(e * (1.0 + g_ref[...]) + b_ref[...], 0.0)


def _mm_film(x, w, g, b):
    m, k = x.shape
    n = w.shape[1]
    return pl.pallas_call(
        _mm_film_body,
        out_shape=jax.ShapeDtypeStruct((m, n), F32),
    )(x, w, g, b)


def _ffn_body(f_ref, w1_ref, b1_ref, w2_ref, b2_ref, o_ref, acc_ref):
    i = pl.program_id(0)

    @pl.when(i == 0)
    def _():
        acc_ref[...] = jnp.zeros_like(acc_ref)

    acc_ref[...] += jnp.dot(f_ref[...], w1_ref[...], preferred_element_type=F32)

    @pl.when(i == 5)
    def _():
        hdn = jnp.maximum(acc_ref[...] + b1_ref[...], 0.0)
        o_ref[...] = jnp.dot(hdn, w2_ref[...], preferred_element_type=F32) + b2_ref[...]


def _ffn(feats, W1, b1r, W2, b2r):
    kb = 13056 // 6
    return pl.pallas_call(
        _ffn_body,
        grid=(6,),
        in_specs=[
            pl.BlockSpec((8, kb), lambda i: (0, i)),
            pl.BlockSpec((kb, 1024), lambda i: (i, 0)),
            pl.BlockSpec((1, 1024), lambda i: (0, 0)),
            pl.BlockSpec((1024, 3129), lambda i: (0, 0)),
            pl.BlockSpec((1, 3129), lambda i: (0, 0)),
        ],
        out_specs=pl.BlockSpec((8, 3129), lambda i: (0, 0)),
        out_shape=jax.ShapeDtypeStruct((8, 3129), F32),
        scratch_shapes=[pltpu.VMEM((8, 1024), F32)],
    )(feats, W1, b1r, W2, b2r)


def kernel(imgs, words, table, Wih_f, Whh_f, b_f, Wih_b, Whh_b, b_b, Wu1,
           Wfu, bfu, Wu2, We1, We2, We3, Wfe, bfe, W1, b1, W2, b2):
    idxp = jnp.zeros((256,), jnp.int32).at[:160].set(
        words.T.reshape(-1).astype(jnp.int32))
    rows = _sc_gather(table, idxp)
    x = rows[:160]

    bcat = jnp.concatenate([b_f, b_b]).reshape(1, 2048)
    question, gb, gb2 = _lstm(
        x, Wih_f, Wih_b, bcat, Whh_f, Whh_b,
        Wfu, bfu.reshape(1, 64), Wfe, bfe.reshape(1, 512))
    gamma3 = gb[:, :32].reshape(8, 1, 32)
    beta3 = gb[:, 32:].reshape(8, 1, 32)
    g2 = gb2[:, :256]
    bt2 = gb2[:, 256:]

    ph = imgs.reshape(8, 3, 112, 2, 112, 2).transpose(0, 1, 3, 5, 2, 4)
    h1 = _conv1(ph, Wu1.reshape(32, 27), gamma3, beta3)

    w_ph, masked_ph = _updown(h1, ph, Wu2.reshape(32, 9))
    weights = w_ph.transpose(0, 3, 1, 4, 2).reshape(8, 1, 224, 224)

    bf16 = jnp.bfloat16
    mp = masked_ph.astype(bf16).reshape(8, 3, 2, 2, 56, 2, 56, 2)
    patches = mp.transpose(0, 4, 6, 5, 2, 7, 3, 1).reshape(25088, 48)
    e1 = _mm_relu_grid(patches, We1.transpose(2, 3, 1, 0).reshape(48, 64).astype(bf16), 3136)

    p2 = e1.astype(bf16).reshape(8, 14, 4, 14, 4, 64).transpose(0, 1, 3, 2, 4, 5).reshape(1568, 1024)
    e2 = _mm_relu_grid(p2, We2.transpose(2, 3, 1, 0).reshape(1024, 128).astype(bf16), 1568)

    p3 = e2.astype(bf16).reshape(8, 7, 2, 7, 2, 128).transpose(0, 1, 3, 2, 4, 5).reshape(392, 512)
    e3 = _mm_film(p3, We3.transpose(2, 3, 1, 0).reshape(512, 256).astype(bf16),
                  jnp.repeat(g2, 49, axis=0), jnp.repeat(bt2, 49, axis=0))

    img_embed = e3.reshape(8, 49, 256).transpose(0, 2, 1).reshape(8, 12544)
    feats = jnp.concatenate([img_embed, question], axis=1)
    s = _ffn(feats, W1, b1.reshape(1, 1024), W2, b2.reshape(1, 3129))
    return s, weights


# speedup vs baseline: 1.6374x; 1.2528x over previous
import functools

import jax
import jax.numpy as jnp
from jax import lax
from jax.experimental import pallas as pl
from jax.experimental.pallas import tpu as pltpu
from jax.experimental.pallas import tpu_sc as plsc

F32 = jnp.float32


def _bf(v):
    return v.astype(jnp.bfloat16).astype(F32)


def _sc_gather(table, idxp):
    info = plsc.get_sparse_core_info()
    nc, ns = info.num_cores, info.num_subcores
    nw = nc * ns
    bpw = 256 // nw
    mesh = plsc.VectorSubcoreMesh(core_axis_name="c", subcore_axis_name="s")

    @functools.partial(
        pl.kernel, mesh=mesh,
        out_type=jax.ShapeDtypeStruct((256, 512), F32),
        scratch_types=[
            pltpu.VMEM((bpw,), jnp.int32),
            pltpu.VMEM((bpw, 512), F32),
            pltpu.SemaphoreType.DMA,
        ],
    )
    def k(table_hbm, idx_hbm, out_hbm, idx_v, rows_v, sem):
        wid = lax.axis_index("s") * nc + lax.axis_index("c")
        base = wid * bpw
        pltpu.sync_copy(idx_hbm.at[pl.ds(base, bpw)], idx_v)
        pltpu.async_copy(table_hbm.at[idx_v], rows_v, sem).wait()
        pltpu.sync_copy(rows_v, out_hbm.at[pl.ds(base, bpw)])

    return k(table, idxp)


def _lstm_body(x_ref, wihf_ref, wihb_ref, bcat_ref, whhf_ref, whhb_ref,
               wfu_ref, bfu_ref, wfe_ref, bfe_ref,
               q_ref, gb_ref, gb2_ref, xw_ref):
    x = x_ref[...]
    dn = (((1,), (1,)), ((), ()))
    xwf = lax.dot_general(x, wihf_ref[...], dn, preferred_element_type=F32)
    xwb = lax.dot_general(x, wihb_ref[...], dn, preferred_element_type=F32)
    xw_ref[...] = jnp.concatenate([xwf, xwb], axis=1) + bcat_ref[...]

    whhf = whhf_ref[...]
    whhb = whhb_ref[...]

    def step(t, carry):
        hf, cf, hb, cb, sf, sb = carry
        gf = xw_ref[pl.ds(t * 8, 8), :1024] + lax.dot_general(
            hf, whhf, dn, preferred_element_type=F32)
        i = jax.nn.sigmoid(gf[:, :256])
        f = jax.nn.sigmoid(gf[:, 256:512])
        g = jnp.tanh(gf[:, 512:768])
        o = jax.nn.sigmoid(gf[:, 768:])
        cf = f * cf + i * g
        hf = o * jnp.tanh(cf)
        gb_ = xw_ref[pl.ds((19 - t) * 8, 8), 1024:] + lax.dot_general(
            hb, whhb, dn, preferred_element_type=F32)
        i = jax.nn.sigmoid(gb_[:, :256])
        f = jax.nn.sigmoid(gb_[:, 256:512])
        g = jnp.tanh(gb_[:, 512:768])
        o = jax.nn.sigmoid(gb_[:, 768:])
        cb = f * cb + i * g
        hb = o * jnp.tanh(cb)
        return hf, cf, hb, cb, sf + hf, sb + hb

    z = jnp.zeros((8, 256), F32)
    hf, cf, hb, cb, sf, sb = lax.fori_loop(0, 20, step, (z, z, z, z, z, z))
    q_ref[...] = jnp.concatenate([hf, hb], axis=1)
    cond = jnp.concatenate([sf, sb], axis=1) * (1.0 / 20.0)
    gb_ref[...] = jnp.dot(cond, wfu_ref[...], preferred_element_type=F32) + bfu_ref[...]
    gb2_ref[...] = jnp.dot(cond, wfe_ref[...], preferred_element_type=F32) + bfe_ref[...]


def _lstm(x, Wih_f, Wih_b, bcat, Whh_f, Whh_b, Wfu, bfu2, Wfe, bfe2):
    return pl.pallas_call(
        _lstm_body,
        out_shape=(
            jax.ShapeDtypeStruct((8, 512), F32),
            jax.ShapeDtypeStruct((8, 64), F32),
            jax.ShapeDtypeStruct((8, 512), F32),
        ),
        scratch_shapes=[pltpu.VMEM((160, 2048), F32)],
    )(x, Wih_f, Wih_b, bcat, Whh_f, Whh_b, Wfu, bfu2, Wfe, bfe2)


def _sel_a(off):
    c = lax.broadcasted_iota(jnp.int32, (224, 112), 0)
    w = lax.broadcasted_iota(jnp.int32, (224, 112), 1)
    return (c == 2 * w + off).astype(F32)


def _sel_b(off):
    r = lax.broadcasted_iota(jnp.int32, (112, 224), 1)
    h = lax.broadcasted_iota(jnp.int32, (112, 224), 0)
    return (r == 2 * h + off).astype(F32)


def _conv1_body(img_ref, w_ref, gamma_ref, beta_ref, h1_ref):
    wb = _bf(w_ref[...])
    sa = [_sel_a(0), _sel_a(1), _sel_a(2)]
    sb = [_sel_b(0), _sel_b(1), _sel_b(2)]
    acc = jnp.zeros((32, 112, 112), F32)
    for ci in range(3):
        img = img_ref[0, ci]
        for dy in range(3):
            r = jnp.dot(sb[dy], img, preferred_element_type=F32)
            for dx in range(3):
                pln = _bf(jnp.dot(r, sa[dx], preferred_element_type=F32))
                wv = wb[:, ci * 9 + dy * 3 + dx].reshape(32, 1, 1)
                acc = acc + wv * pln[None]
    acc = jnp.maximum(acc, 0.0)
    gamma = gamma_ref[0].reshape(32, 1, 1)
    beta = beta_ref[0].reshape(32, 1, 1)
    h1_ref[0] = acc * (1.0 + gamma) + beta


def _conv1(imgs, w1r, gamma3, beta3):
    return pl.pallas_call(
        _conv1_body,
        grid=(8,),
        in_specs=[
            pl.BlockSpec((1, 3, 224, 224), lambda b: (b, 0, 0, 0)),
            pl.BlockSpec((32, 27), lambda b: (0, 0)),
            pl.BlockSpec((1, 1, 32), lambda b: (b, 0, 0)),
            pl.BlockSpec((1, 1, 32), lambda b: (b, 0, 0)),
        ],
        out_specs=pl.BlockSpec((1, 32, 112, 112), lambda b: (b, 0, 0, 0)),
        out_shape=jax.ShapeDtypeStruct((8, 32, 112, 112), F32),
    )(imgs, w1r, gamma3, beta3)


_CMAP = {0: ((1, -1), (0, 0), (1, 0)), 1: ((0, 0), (1, 0), (0, 1))}


def _shift(p, sy, sx):
    if sy == 1:
        p = jnp.concatenate([p[..., 1:, :], jnp.zeros_like(p[..., :1, :])], axis=-2)
    elif sy == -1:
        p = jnp.concatenate([jnp.zeros_like(p[..., :1, :]), p[..., :-1, :]], axis=-2)
    if sx == 1:
        p = jnp.concatenate([p[..., :, 1:], jnp.zeros_like(p[..., :, :1])], axis=-1)
    elif sx == -1:
        p = jnp.concatenate([jnp.zeros_like(p[..., :, :1]), p[..., :, :-1]], axis=-1)
    return p


def _updown_body(h1_ref, img_ref, w_ref, w224_ref, mph_ref):
    x = h1_ref[0]
    rsh = jnp.concatenate([x[:, :, :1], x[:, :, :-1]], axis=2)
    lsh = jnp.concatenate([x[:, :, 1:], x[:, :, -1:]], axis=2)
    upc = (0.75 * x + 0.25 * rsh, 0.75 * x + 0.25 * lsh)
    up = {}
    for px in range(2):
        u = upc[px]
        rshr = jnp.concatenate([u[:, :1, :], u[:, :-1, :]], axis=1)
        lshr = jnp.concatenate([u[:, 1:, :], u[:, -1:, :]], axis=1)
        up[(0, px)] = _bf(0.75 * u + 0.25 * rshr)
        up[(1, px)] = _bf(0.75 * u + 0.25 * lshr)
    wb = _bf(w_ref[...])
    sa = (_sel_a(0), _sel_a(1))
    sb = (_sel_b(0), _sel_b(1))
    w224 = jnp.zeros((224, 224), F32)
    mask = {}
    for py in range(2):
        for px in range(2):
            acc = jnp.zeros((32, 112, 112), F32)
            for dy in range(3):
                ry, sy = _CMAP[py][dy]
                for dx in range(3):
                    rx, sx = _CMAP[px][dx]
                    pln = _shift(up[(ry, rx)], sy, sx)
                    acc = acc + wb[:, dy * 3 + dx].reshape(32, 1, 1) * pln
            wpl = jnp.sum(acc, axis=0)
            mask[(py, px)] = (wpl > 0.0).astype(F32)
            w224 = w224 + jnp.dot(
                sa[py], jnp.dot(wpl, sb[px], preferred_element_type=F32),
                preferred_element_type=F32)
    w224_ref[0, 0] = w224
    for ci in range(3):
        img = img_ref[0, ci]
        for py in range(2):
            r = jnp.dot(sb[py], img, preferred_element_type=F32)
            for px in range(2):
                phase = jnp.dot(r, sa[px], preferred_element_type=F32)
                mph_ref[0, ci, py, px] = (phase * mask[(py, px)]).astype(jnp.bfloat16)


def _updown(h1, imgs, w2r):
    return pl.pallas_call(
        _updown_body,
        grid=(8,),
        in_specs=[
            pl.BlockSpec((1, 32, 112, 112), lambda b: (b, 0, 0, 0)),
            pl.BlockSpec((1, 3, 224, 224), lambda b: (b, 0, 0, 0)),
            pl.BlockSpec((32, 9), lambda b: (0, 0)),
        ],
        out_specs=(
            pl.BlockSpec((1, 1, 224, 224), lambda b: (b, 0, 0, 0)),
            pl.BlockSpec((1, 3, 2, 2, 112, 112), lambda b: (b, 0, 0, 0, 0, 0)),
        ),
        out_shape=(
            jax.ShapeDtypeStruct((8, 1, 224, 224), F32),
            jax.ShapeDtypeStruct((8, 3, 2, 2, 112, 112), jnp.bfloat16),
        ),
    )(h1, imgs, w2r)


def _mm_relu_body(x_ref, w_ref, o_ref):
    o_ref[...] = jnp.maximum(
        jnp.dot(x_ref[...], w_ref[...], preferred_element_type=F32), 0.0)


def _mm_relu_grid(x, w, mblk):
    m, k = x.shape
    n = w.shape[1]
    return pl.pallas_call(
        _mm_relu_body,
        grid=(m // mblk,),
        in_specs=[
            pl.BlockSpec((mblk, k), lambda i: (i, 0)),
            pl.BlockSpec((k, n), lambda i: (0, 0)),
        ],
        out_specs=pl.BlockSpec((mblk, n), lambda i: (i, 0)),
        out_shape=jax.ShapeDtypeStruct((m, n), F32),
    )(x, w)


def _mm_film_body(x_ref, w_ref, g_ref, b_ref, o_ref):
    e = jnp.dot(x_ref[...], w_ref[...], preferred_element_type=F32)
    o_ref[...] = jnp.maximum(e * (1.0 + g_ref[...]) + b_ref[...], 0.0)


def _mm_film(x, w, g, b):
    m, k = x.shape
    n = w.shape[1]
    return pl.pallas_call(
        _mm_film_body,
        out_shape=jax.ShapeDtypeStruct((m, n), F32),
    )(x, w, g, b)


def _ffn_body(f_ref, w1_ref, b1_ref, w2_ref, b2_ref, o_ref, acc_ref):
    i = pl.program_id(0)

    @pl.when(i == 0)
    def _():
        acc_ref[...] = jnp.zeros_like(acc_ref)

    acc_ref[...] += jnp.dot(f_ref[...], w1_ref[...], preferred_element_type=F32)

    @pl.when(i == 5)
    def _():
        hdn = jnp.maximum(acc_ref[...] + b1_ref[...], 0.0)
        o_ref[...] = jnp.dot(hdn, w2_ref[...], preferred_element_type=F32) + b2_ref[...]


def _ffn(feats, W1, b1r, W2, b2r):
    kb = 13056 // 6
    return pl.pallas_call(
        _ffn_body,
        grid=(6,),
        in_specs=[
            pl.BlockSpec((8, kb), lambda i: (0, i)),
            pl.BlockSpec((kb, 1024), lambda i: (i, 0)),
            pl.BlockSpec((1, 1024), lambda i: (0, 0)),
            pl.BlockSpec((1024, 3129), lambda i: (0, 0)),
            pl.BlockSpec((1, 3129), lambda i: (0, 0)),
        ],
        out_specs=pl.BlockSpec((8, 3129), lambda i: (0, 0)),
        out_shape=jax.ShapeDtypeStruct((8, 3129), F32),
        scratch_shapes=[pltpu.VMEM((8, 1024), F32)],
    )(feats, W1, b1r, W2, b2r)


def kernel(imgs, words, table, Wih_f, Whh_f, b_f, Wih_b, Whh_b, b_b, Wu1,
           Wfu, bfu, Wu2, We1, We2, We3, Wfe, bfe, W1, b1, W2, b2):
    idxp = jnp.zeros((256,), jnp.int32).at[:160].set(
        words.T.reshape(-1).astype(jnp.int32))
    rows = _sc_gather(table, idxp)
    x = rows[:160]

    bcat = jnp.concatenate([b_f, b_b]).reshape(1, 2048)
    question, gb, gb2 = _lstm(
        x, Wih_f, Wih_b, bcat, Whh_f, Whh_b,
        Wfu, bfu.reshape(1, 64), Wfe, bfe.reshape(1, 512))
    gamma3 = gb[:, :32].reshape(8, 1, 32)
    beta3 = gb[:, 32:].reshape(8, 1, 32)
    g2 = gb2[:, :256]
    bt2 = gb2[:, 256:]

    h1 = _conv1(imgs, Wu1.reshape(32, 27), gamma3, beta3)

    weights, masked_ph = _updown(h1, imgs, Wu2.reshape(32, 9))

    bf16 = jnp.bfloat16
    mp = masked_ph.reshape(8, 3, 2, 2, 56, 2, 56, 2)
    patches = mp.transpose(0, 4, 6, 5, 2, 7, 3, 1).reshape(25088, 48)
    e1 = _mm_relu_grid(patches, We1.transpose(2, 3, 1, 0).reshape(48, 64).astype(bf16), 3136)

    p2 = e1.astype(bf16).reshape(8, 14, 4, 14, 4, 64).transpose(0, 1, 3, 2, 4, 5).reshape(1568, 1024)
    e2 = _mm_relu_grid(p2, We2.transpose(2, 3, 1, 0).reshape(1024, 128).astype(bf16), 1568)

    p3 = e2.astype(bf16).reshape(8, 7, 2, 7, 2, 128).transpose(0, 1, 3, 2, 4, 5).reshape(392, 512)
    e3 = _mm_film(p3, We3.transpose(2, 3, 1, 0).reshape(512, 256).astype(bf16),
                  jnp.repeat(g2, 49, axis=0), jnp.repeat(bt2, 49, axis=0))

    img_embed = e3.reshape(8, 49, 256).transpose(0, 2, 1).reshape(8, 12544)
    feats = jnp.concatenate([img_embed, question], axis=1)
    s = _ffn(feats, W1, b1.reshape(1, 1024), W2, b2.reshape(1, 3129))
    return s, weights
